# baseline (speedup 1.0000x reference)
import functools

import jax
import jax.numpy as jnp
from jax import lax
from jax.experimental import pallas as pl
from jax.experimental.pallas import tpu as pltpu
from jax.experimental.pallas import tpu_sc as plsc

_CHUNK = 128


@functools.lru_cache(maxsize=None)
def _make_gather_kernel(B: int, D: int, n_users: int, n_items: int):
    info = plsc.get_sparse_core_info()
    nw = info.num_cores * info.num_subcores
    per_w = B // nw
    n_chunk = per_w // _CHUNK

    mesh = plsc.VectorSubcoreMesh(core_axis_name="c", subcore_axis_name="s")
    out = jax.ShapeDtypeStruct((nw, n_chunk, _CHUNK, D), jnp.float32)
    idx_t = pltpu.VMEM((n_chunk, _CHUNK), jnp.int32)
    rows_t = pltpu.VMEM((n_chunk, _CHUNK, D), jnp.float32)

    @functools.partial(
        pl.kernel,
        mesh=mesh,
        out_type=(out, out, out),
        scratch_types=[idx_t, idx_t, idx_t, rows_t, rows_t, rows_t,
                       pltpu.SemaphoreType.DMA],
        compiler_params=pltpu.CompilerParams(use_tc_tiling_on_sc=False),
    )
    def gather3(u_ix, p_ix, n_ix, uemb, iemb, out_u, out_p, out_n,
                idx_u, idx_p, idx_n, rows_u, rows_p, rows_n, sem):
        wid = lax.axis_index("s") * info.num_cores + lax.axis_index("c")
        pltpu.sync_copy(u_ix.at[wid], idx_u)
        pltpu.sync_copy(p_ix.at[wid], idx_p)
        pltpu.sync_copy(n_ix.at[wid], idx_n)
        copies = []
        for idx, tab, rows in ((idx_u, uemb, rows_u),
                               (idx_p, iemb, rows_p),
                               (idx_n, iemb, rows_n)):
            for j in range(n_chunk):
                copies.append(pltpu.async_copy(tab.at[idx.at[j]], rows.at[j], sem))
        for cp in copies:
            cp.wait()
        pltpu.sync_copy(rows_u, out_u.at[wid])
        pltpu.sync_copy(rows_p, out_p.at[wid])
        pltpu.sync_copy(rows_n, out_n.at[wid])

    return gather3, nw, n_chunk


def kernel(users, positives, negatives, User_Emb, Item_Emb):
    B = users.shape[0]
    D = User_Emb.shape[1]
    gather3, nw, n_chunk = _make_gather_kernel(
        B, D, User_Emb.shape[0], Item_Emb.shape[0])
    shape3 = (nw, n_chunk, _CHUNK)
    u = users.astype(jnp.int32).reshape(shape3)
    p = positives.astype(jnp.int32).reshape(shape3)
    n = negatives.astype(jnp.int32).reshape(shape3)
    out_u, out_p, out_n = gather3(u, p, n, User_Emb, Item_Emb)
    return (out_u.reshape(B, D), out_p.reshape(B, D), out_n.reshape(B, D))
